# Initial kernel scaffold; baseline (speedup 1.0000x reference)
import jax
import jax.numpy as jnp
from jax.experimental import pallas as pl


def kernel(ranks, mu, w):
    raise NotImplementedError("write your pallas kernel here")


# speedup vs baseline: 7.5429x; 7.5429x over previous
import functools

import jax
import jax.numpy as jnp
from jax import lax
from jax.experimental import pallas as pl
from jax.experimental.pallas import tpu as pltpu

_VOCAB = 100000
_N = 4096
_K = 25
_DX = 64
_DY = 64
_LO = -4.0
_HI = 4.0

_NB = 8


def _field_body(mux_ref, muy_ref, w_ref, out_ref):
    mux = mux_ref[...]
    muy = muy_ref[...]
    wv = w_ref[...]
    step = (_HI - _LO) / (_DX - 1)
    gi = lax.broadcasted_iota(jnp.int32, (1, _DX), 1).astype(jnp.float32) * step + _LO
    acc = jnp.zeros((_NB, _DX, _DY), jnp.float32)
    for k in range(_K):
        ax = jnp.exp(-0.5 * (gi - mux[:, k:k + 1]) ** 2)
        by = jnp.exp(-0.5 * (gi - muy[:, k:k + 1]) ** 2) * wv[:, k:k + 1]
        acc = acc + ax[:, :, None] * by[:, None, :]
    out_ref[...] = acc


def _field(mux, muy, wv, interpret=False):
    grid = _N // _NB
    return pl.pallas_call(
        _field_body,
        grid=(grid,),
        in_specs=[
            pl.BlockSpec((_NB, _K), lambda i: (i, 0)),
            pl.BlockSpec((_NB, _K), lambda i: (i, 0)),
            pl.BlockSpec((_NB, _K), lambda i: (i, 0)),
        ],
        out_specs=pl.BlockSpec((_NB, _DX, _DY), lambda i: (i, 0, 0)),
        out_shape=jax.ShapeDtypeStruct((_N, _DX, _DY), jnp.float32),
        interpret=interpret,
    )(mux, muy, wv)


def kernel(ranks, mu, w):
    mu_s = jnp.take(mu, ranks, axis=0)
    w_s = jnp.take(w, ranks, axis=0)
    mux = mu_s[:, :, 0]
    muy = mu_s[:, :, 1]
    out = _field(mux, muy, w_s)
    return out.reshape(_N, _DX * _DY)


# speedup vs baseline: 22.6802x; 3.0068x over previous
import functools

import jax
import jax.numpy as jnp
from jax import lax
from jax.experimental import pallas as pl
from jax.experimental.pallas import tpu as pltpu

_VOCAB = 100000
_N = 4096
_K = 25
_KP = 32
_DX = 64
_DY = 64
_LO = -4.0
_HI = 4.0
_STEP = (_HI - _LO) / (_DX - 1)

_NB = 8
_CW = _NB * _KP
_RW = _NB * _DX


def _field_body(mask_ref, mux_ref, muy_ref, w_ref, out_ref):
    mask = mask_ref[...]
    mux = mux_ref[0]
    muy = muy_ref[0]
    wv = w_ref[0]
    gx = lax.broadcasted_iota(jnp.int32, (_DX, 1), 0).astype(jnp.float32) * _STEP + _LO
    ax = jnp.exp(-0.5 * (gx - mux) ** 2)
    a_bd = jnp.concatenate([ax] * _NB, axis=0) * mask
    gy = lax.broadcasted_iota(jnp.int32, (1, _DY), 1).astype(jnp.float32) * _STEP + _LO
    b = jnp.exp(-0.5 * (gy - muy) ** 2) * wv
    out_ref[...] = jnp.dot(a_bd, b, preferred_element_type=jnp.float32)


def _field(mux_p, muy_p, w_p, mask, interpret=False):
    grid = _N // _NB
    return pl.pallas_call(
        _field_body,
        grid=(grid,),
        in_specs=[
            pl.BlockSpec((_RW, _CW), lambda i: (0, 0)),
            pl.BlockSpec((1, 1, _CW), lambda i: (i, 0, 0)),
            pl.BlockSpec((1, _CW, 1), lambda i: (i, 0, 0)),
            pl.BlockSpec((1, _CW, 1), lambda i: (i, 0, 0)),
        ],
        out_specs=pl.BlockSpec((_RW, _DY), lambda i: (i, 0)),
        out_shape=jax.ShapeDtypeStruct((_N * _DX, _DY), jnp.float32),
        interpret=interpret,
    )(mask, mux_p, muy_p, w_p)


def kernel(ranks, mu, w):
    mu_s = jnp.take(mu, ranks, axis=0)
    w_s = jnp.take(w, ranks, axis=0)
    mux = mu_s[:, :, 0]
    muy = mu_s[:, :, 1]

    g = _N // _NB
    pad = ((0, 0), (0, _KP - _K))
    mux_p = jnp.pad(mux, pad).reshape(g, 1, _CW)
    muy_p = jnp.pad(muy, pad).reshape(g, _CW, 1)
    w_p = jnp.pad(w_s, pad).reshape(g, _CW, 1)

    r_i = lax.broadcasted_iota(jnp.int32, (_RW, _CW), 0)
    c_i = lax.broadcasted_iota(jnp.int32, (_RW, _CW), 1)
    mask = (r_i // _DX == c_i // _KP).astype(jnp.float32)

    out = _field(mux_p, muy_p, w_p, mask)
    return out.reshape(_N, _DX * _DY)


# speedup vs baseline: 36.4724x; 1.6081x over previous
import functools

import jax
import jax.numpy as jnp
from jax import lax
from jax.experimental import pallas as pl
from jax.experimental.pallas import tpu as pltpu

_VOCAB = 100000
_N = 4096
_K = 25
_KP = 32
_DX = 64
_DY = 64
_LO = -4.0
_HI = 4.0
_STEP = (_HI - _LO) / (_DX - 1)

_NB = 8
_CW = _NB * _KP
_RW = _NB * _DX


_GP = 8


def _field_body(mux_ref, muy_ref, w_ref, out_ref):
    gx = lax.broadcasted_iota(jnp.int32, (_DX, 1), 0).astype(jnp.float32) * _STEP + _LO
    gy = lax.broadcasted_iota(jnp.int32, (1, _DY), 1).astype(jnp.float32) * _STEP + _LO
    r_i = lax.broadcasted_iota(jnp.int32, (_RW, 1), 0)
    c_i = lax.broadcasted_iota(jnp.int32, (1, _CW), 1)
    mask = (r_i // _DX) == (c_i // _KP)
    for g in range(_GP):
        mux = mux_ref[g]
        muy = muy_ref[g]
        wv = w_ref[g]
        ax = jnp.exp(-0.5 * (gx - mux) ** 2)
        tiled = jnp.concatenate([ax] * _NB, axis=0)
        a_bd = jnp.where(mask, tiled, 0.0).astype(jnp.bfloat16)
        b = (jnp.exp(-0.5 * (gy - muy) ** 2) * wv).astype(jnp.bfloat16)
        out_ref[pl.ds(g * _RW, _RW), :] = jnp.dot(
            a_bd, b, preferred_element_type=jnp.float32)


def _field(mux_p, muy_p, w_p, interpret=False):
    grid = _N // (_NB * _GP)
    return pl.pallas_call(
        _field_body,
        grid=(grid,),
        in_specs=[
            pl.BlockSpec((_GP, 1, _CW), lambda i: (i, 0, 0)),
            pl.BlockSpec((_GP, _CW, 1), lambda i: (i, 0, 0)),
            pl.BlockSpec((_GP, _CW, 1), lambda i: (i, 0, 0)),
        ],
        out_specs=pl.BlockSpec((_GP * _RW, _DY), lambda i: (i, 0)),
        out_shape=jax.ShapeDtypeStruct((_N * _DX, _DY), jnp.float32),
        interpret=interpret,
    )(mux_p, muy_p, w_p)


def kernel(ranks, mu, w):
    mu_s = jnp.take(mu, ranks, axis=0)
    w_s = jnp.take(w, ranks, axis=0)
    mux = mu_s[:, :, 0]
    muy = mu_s[:, :, 1]

    g = _N // _NB
    pad = ((0, 0), (0, _KP - _K))
    mux_p = jnp.pad(mux, pad).reshape(g, 1, _CW)
    muy_p = jnp.pad(muy, pad).reshape(g, _CW, 1)
    w_p = jnp.pad(w_s, pad).reshape(g, _CW, 1)

    out = _field(mux_p, muy_p, w_p)
    return out.reshape(_N, _DX * _DY)
